# baseline (speedup 1.0000x reference)
import functools

import jax
import jax.numpy as jnp
from jax import lax
from jax.experimental import pallas as pl
from jax.experimental.pallas import tpu as pltpu
from jax.experimental.pallas import tpu_sc as plsc

VOCAB = 100000
EMBED = 128
HIDDEN = 1024
LABELS = 1000
B = 16384
L = 20

NC = 2
NS = 16
NW = NC * NS
BPW = B // NW
CB = 4
ROWS = CB * L
NCHUNK = BPW // CB


NBUF = 2
NSLICE = 4
SB = B // NSLICE
SBPW = SB // NW
SNCHUNK = SBPW // CB


def _sc_body(emb_hbm, ids_hbm, out_hbm, idx_v, buf0, buf1,
             stage_v, sem0, sem1):
    bufs = (buf0, buf1)
    sems = (sem0, sem1)
    wid = lax.axis_index("s") * NC + lax.axis_index("c")
    pltpu.sync_copy(ids_hbm.at[pl.ds(wid * SNCHUNK, SNCHUNK), :], idx_v)

    for j in range(NBUF):
        pltpu.async_copy(emb_hbm.at[idx_v.at[j]], bufs[j], sems[j])

    def pool(c, buf):
        for t in range(CB):
            for q in range(EMBED // 16):
                col = pl.ds(q * 16, 16)
                vals = [buf[t * L + r, col] for r in range(L)]
                while len(vals) > 1:
                    nxt = [a + b for a, b in zip(vals[0::2], vals[1::2])]
                    if len(vals) % 2:
                        nxt.append(vals[-1])
                    vals = nxt
                stage_v[c * CB + t, col] = vals[0]

    def step(k, carry):
        for j in range(NBUF):
            c = NBUF * k + j
            pltpu.make_async_copy(emb_hbm.at[idx_v.at[c]], bufs[j], sems[j]).wait()
            pool(c, bufs[j])

            @pl.when(k < SNCHUNK // NBUF - 1)
            def _():
                pltpu.async_copy(emb_hbm.at[idx_v.at[c + NBUF]], bufs[j], sems[j])

        return carry

    lax.fori_loop(0, SNCHUNK // NBUF, step, 0)
    pltpu.sync_copy(stage_v, out_hbm.at[pl.ds(wid * SBPW, SBPW), :])


def _sc_gather_pool(emb, ids_slice):
    mesh = plsc.VectorSubcoreMesh(core_axis_name="c", subcore_axis_name="s")
    f = pl.kernel(
        _sc_body,
        mesh=mesh,
        out_type=jax.ShapeDtypeStruct((SB, EMBED), jnp.float32),
        scratch_types=(
            [pltpu.VMEM((SNCHUNK, ROWS), jnp.int32)]
            + [pltpu.VMEM((ROWS, EMBED), jnp.float32)] * NBUF
            + [pltpu.VMEM((SBPW, EMBED), jnp.float32)]
            + [pltpu.SemaphoreType.DMA] * NBUF
        ),
    )
    return f(emb, ids_slice)


BB = 2048


def _collapse_body(w1_ref, b1_ref, w2_ref, b2_ref, wct_ref, bc_ref):
    wct = lax.dot_general(
        w2_ref[...], w1_ref[...], (((1,), (0,)), ((), ())),
        preferred_element_type=jnp.float32)
    wct_ref[...] = wct * (1.0 / L)
    bc = lax.dot_general(
        b1_ref[...], w2_ref[...], (((1,), (1,)), ((), ())),
        preferred_element_type=jnp.float32)
    bc_ref[...] = bc + b2_ref[...]


def _collapse(W1, b1, W2, b2):
    return pl.pallas_call(
        _collapse_body,
        out_shape=(
            jax.ShapeDtypeStruct((LABELS, EMBED), jnp.float32),
            jax.ShapeDtypeStruct((1, LABELS), jnp.float32),
        ),
    )(W1, b1.reshape(1, HIDDEN), W2, b2.reshape(1, LABELS))


def _tc_body(x_ref, wct_ref, bc_ref, o_ref):
    logits = lax.dot_general(
        x_ref[...], wct_ref[...], (((1,), (1,)), ((), ())),
        preferred_element_type=jnp.float32) + bc_ref[...]
    m = jnp.max(logits, axis=1, keepdims=True)
    s = logits - m
    o_ref[...] = s - jnp.log(jnp.sum(jnp.exp(s), axis=1, keepdims=True))


def _tc_body_alias(prev_ref, x_ref, wct_ref, bc_ref, o_ref):
    del prev_ref
    _tc_body(x_ref, wct_ref, bc_ref, o_ref)


def _tc_mlp_slice(prev, seq_slice, wct, bc, slice_idx):
    nblk = SB // BB
    x_spec = pl.BlockSpec((BB, EMBED), lambda j: (j, 0))
    w_spec = pl.BlockSpec((LABELS, EMBED), lambda j: (0, 0))
    b_spec = pl.BlockSpec((1, LABELS), lambda j: (0, 0))
    out_spec = pl.BlockSpec(
        (BB, LABELS), lambda j, s=slice_idx: (s * nblk + j, 0))
    out_shape = jax.ShapeDtypeStruct((B, LABELS), jnp.float32)
    if prev is None:
        return pl.pallas_call(
            _tc_body,
            grid=(nblk,),
            in_specs=[x_spec, w_spec, b_spec],
            out_specs=out_spec,
            out_shape=out_shape,
        )(seq_slice, wct, bc)
    return pl.pallas_call(
        _tc_body_alias,
        grid=(nblk,),
        in_specs=[
            pl.BlockSpec(memory_space=pltpu.MemorySpace.HBM),
            x_spec, w_spec, b_spec,
        ],
        out_specs=out_spec,
        out_shape=out_shape,
        input_output_aliases={0: 0},
    )(prev, seq_slice, wct, bc)


@jax.jit
def kernel(input_ids, seq_len, emb, W1, b1, W2, b2):
    ids2 = input_ids.astype(jnp.int32).reshape(B // CB, ROWS)
    wct, bc = _collapse(W1, b1, W2, b2)
    rows_per_slice = SB // CB
    seqs = [
        _sc_gather_pool(
            emb, lax.slice(ids2, (i * rows_per_slice, 0),
                           ((i + 1) * rows_per_slice, ROWS)))
        for i in range(NSLICE)
    ]
    out = None
    for i in range(NSLICE):
        out = _tc_mlp_slice(out, seqs[i], wct, bc, i)
    return out


# speedup vs baseline: 1.0039x; 1.0039x over previous
import functools

import jax
import jax.numpy as jnp
from jax import lax
from jax.experimental import pallas as pl
from jax.experimental.pallas import tpu as pltpu
from jax.experimental.pallas import tpu_sc as plsc

VOCAB = 100000
EMBED = 128
HIDDEN = 1024
LABELS = 1000
B = 16384
L = 20

NC = 2
NS = 16
NW = NC * NS
BPW = B // NW
CB = 4
ROWS = CB * L
NCHUNK = BPW // CB


NBUF = 2
NSLICE = 4
SB = B // NSLICE
SBPW = SB // NW
SNCHUNK = SBPW // CB


def _sc_body(emb_hbm, ids_hbm, out_hbm, idx_v, buf0, buf1,
             stage_v, sem0, sem1):
    bufs = (buf0, buf1)
    sems = (sem0, sem1)
    wid = lax.axis_index("s") * NC + lax.axis_index("c")
    pltpu.sync_copy(ids_hbm.at[pl.ds(wid * SNCHUNK, SNCHUNK), :], idx_v)

    for j in range(NBUF):
        pltpu.async_copy(emb_hbm.at[idx_v.at[j]], bufs[j], sems[j])

    def pool(c, buf):
        for t in range(CB):
            for q in range(EMBED // 16):
                col = pl.ds(q * 16, 16)
                vals = [buf[t * L + r, col] for r in range(L)]
                while len(vals) > 1:
                    nxt = [a + b for a, b in zip(vals[0::2], vals[1::2])]
                    if len(vals) % 2:
                        nxt.append(vals[-1])
                    vals = nxt
                stage_v[c * CB + t, col] = vals[0]

    def step(k, carry):
        for j in range(NBUF):
            c = NBUF * k + j
            pltpu.make_async_copy(emb_hbm.at[idx_v.at[c]], bufs[j], sems[j]).wait()
            pool(c, bufs[j])

            @pl.when(k < SNCHUNK // NBUF - 1)
            def _():
                pltpu.async_copy(emb_hbm.at[idx_v.at[c + NBUF]], bufs[j], sems[j])

        return carry

    lax.fori_loop(0, SNCHUNK // NBUF, step, 0)
    pltpu.sync_copy(stage_v, out_hbm.at[pl.ds(wid * SBPW, SBPW), :])


def _sc_gather_pool(emb, ids_slice):
    mesh = plsc.VectorSubcoreMesh(core_axis_name="c", subcore_axis_name="s")
    f = pl.kernel(
        _sc_body,
        mesh=mesh,
        out_type=jax.ShapeDtypeStruct((SB, EMBED), jnp.float32),
        scratch_types=(
            [pltpu.VMEM((SNCHUNK, ROWS), jnp.int32)]
            + [pltpu.VMEM((ROWS, EMBED), jnp.float32)] * NBUF
            + [pltpu.VMEM((SBPW, EMBED), jnp.float32)]
            + [pltpu.SemaphoreType.DMA] * NBUF
        ),
    )
    return f(emb, ids_slice)


BB = 2048


def _collapse_body(w1_ref, b1_ref, w2_ref, b2_ref, wct_ref, bc_ref):
    wct = lax.dot_general(
        w2_ref[...], w1_ref[...], (((1,), (0,)), ((), ())),
        preferred_element_type=jnp.float32)
    wct_ref[...] = wct * (1.0 / L)
    bc = lax.dot_general(
        b1_ref[...], w2_ref[...], (((1,), (1,)), ((), ())),
        preferred_element_type=jnp.float32)
    bc_ref[...] = bc + b2_ref[...]


def _collapse(W1, b1, W2, b2):
    return pl.pallas_call(
        _collapse_body,
        out_shape=(
            jax.ShapeDtypeStruct((LABELS, EMBED), jnp.float32),
            jax.ShapeDtypeStruct((1, LABELS), jnp.float32),
        ),
    )(W1, b1.reshape(1, HIDDEN), W2, b2.reshape(1, LABELS))


def _tc_body(x_ref, wct_ref, bc_ref, o_ref):
    logits = lax.dot_general(
        x_ref[...], wct_ref[...], (((1,), (1,)), ((), ())),
        preferred_element_type=jnp.float32) + bc_ref[...]
    m = jnp.max(logits, axis=1, keepdims=True)
    s = logits - m
    o_ref[...] = s - jnp.log(jnp.sum(jnp.exp(s), axis=1, keepdims=True))


def _tc_body_alias(prev_ref, x_ref, wct_ref, bc_ref, o_ref):
    del prev_ref
    _tc_body(x_ref, wct_ref, bc_ref, o_ref)


def _tc_mlp_slice(prev, seq_slice, wct, bc, slice_idx):
    nblk = SB // BB
    x_spec = pl.BlockSpec((BB, EMBED), lambda j: (j, 0))
    w_spec = pl.BlockSpec((LABELS, EMBED), lambda j: (0, 0))
    b_spec = pl.BlockSpec((1, LABELS), lambda j: (0, 0))
    out_spec = pl.BlockSpec(
        (BB, LABELS), lambda j, s=slice_idx: (s * nblk + j, 0))
    out_shape = jax.ShapeDtypeStruct((B, LABELS), jnp.float32)
    if prev is None:
        return pl.pallas_call(
            _tc_body,
            grid=(nblk,),
            in_specs=[x_spec, w_spec, b_spec],
            out_specs=out_spec,
            out_shape=out_shape,
        )(seq_slice, wct, bc)
    return pl.pallas_call(
        _tc_body_alias,
        grid=(nblk,),
        in_specs=[
            pl.BlockSpec(memory_space=pltpu.MemorySpace.HBM),
            x_spec, w_spec, b_spec,
        ],
        out_specs=out_spec,
        out_shape=out_shape,
        input_output_aliases={0: 0},
    )(prev, seq_slice, wct, bc)


@jax.jit
def kernel(input_ids, seq_len, emb, W1, b1, W2, b2):
    ids2 = input_ids.astype(jnp.int32).reshape(B // CB, ROWS)
    wct, bc = _collapse(W1, b1, W2, b2)
    rows_per_slice = SB // CB

    def sc(i):
        return _sc_gather_pool(
            emb, lax.slice(ids2, (i * rows_per_slice, 0),
                           ((i + 1) * rows_per_slice, ROWS)))

    seq_prev = sc(0)
    out = None
    for i in range(1, NSLICE):
        seq_next = sc(i)
        out = _tc_mlp_slice(out, seq_prev, wct, bc, i - 1)
        seq_prev = seq_next
    return _tc_mlp_slice(out, seq_prev, wct, bc, NSLICE - 1)


# speedup vs baseline: 1.0620x; 1.0580x over previous
import functools

import jax
import jax.numpy as jnp
from jax import lax
from jax.experimental import pallas as pl
from jax.experimental.pallas import tpu as pltpu
from jax.experimental.pallas import tpu_sc as plsc

VOCAB = 100000
EMBED = 128
HIDDEN = 1024
LABELS = 1000
B = 16384
L = 20

NC = 2
NS = 16
NW = NC * NS
BPW = B // NW
CB = 4
ROWS = CB * L
NCHUNK = BPW // CB


NBUF = 2
NSLICE = 2
SB = B // NSLICE
SBPW = SB // NW
SNCHUNK = SBPW // CB


def _sc_body(emb_hbm, ids_hbm, out_hbm, idx_v, buf0, buf1,
             stage_v, sem0, sem1):
    bufs = (buf0, buf1)
    sems = (sem0, sem1)
    wid = lax.axis_index("s") * NC + lax.axis_index("c")
    pltpu.sync_copy(ids_hbm.at[pl.ds(wid * SNCHUNK, SNCHUNK), :], idx_v)

    for j in range(NBUF):
        pltpu.async_copy(emb_hbm.at[idx_v.at[j]], bufs[j], sems[j])

    def pool(c, buf):
        for t in range(CB):
            for q in range(EMBED // 16):
                col = pl.ds(q * 16, 16)
                vals = [buf[t * L + r, col] for r in range(L)]
                while len(vals) > 1:
                    nxt = [a + b for a, b in zip(vals[0::2], vals[1::2])]
                    if len(vals) % 2:
                        nxt.append(vals[-1])
                    vals = nxt
                stage_v[c * CB + t, col] = vals[0]

    def step(k, carry):
        for j in range(NBUF):
            c = NBUF * k + j
            pltpu.make_async_copy(emb_hbm.at[idx_v.at[c]], bufs[j], sems[j]).wait()
            pool(c, bufs[j])

            @pl.when(k < SNCHUNK // NBUF - 1)
            def _():
                pltpu.async_copy(emb_hbm.at[idx_v.at[c + NBUF]], bufs[j], sems[j])

        return carry

    lax.fori_loop(0, SNCHUNK // NBUF, step, 0)
    pltpu.sync_copy(stage_v, out_hbm.at[pl.ds(wid * SBPW, SBPW), :])


def _sc_gather_pool(emb, ids_slice):
    mesh = plsc.VectorSubcoreMesh(core_axis_name="c", subcore_axis_name="s")
    f = pl.kernel(
        _sc_body,
        mesh=mesh,
        out_type=jax.ShapeDtypeStruct((SB, EMBED), jnp.float32),
        scratch_types=(
            [pltpu.VMEM((SNCHUNK, ROWS), jnp.int32)]
            + [pltpu.VMEM((ROWS, EMBED), jnp.float32)] * NBUF
            + [pltpu.VMEM((SBPW, EMBED), jnp.float32)]
            + [pltpu.SemaphoreType.DMA] * NBUF
        ),
    )
    return f(emb, ids_slice)


BB = 2048


def _collapse_body(w1_ref, b1_ref, w2_ref, b2_ref, wct_ref, bc_ref):
    wct = lax.dot_general(
        w2_ref[...], w1_ref[...], (((1,), (0,)), ((), ())),
        preferred_element_type=jnp.float32)
    wct_ref[...] = wct * (1.0 / L)
    bc = lax.dot_general(
        b1_ref[...], w2_ref[...], (((1,), (1,)), ((), ())),
        preferred_element_type=jnp.float32)
    bc_ref[...] = bc + b2_ref[...]


def _collapse(W1, b1, W2, b2):
    return pl.pallas_call(
        _collapse_body,
        out_shape=(
            jax.ShapeDtypeStruct((LABELS, EMBED), jnp.float32),
            jax.ShapeDtypeStruct((1, LABELS), jnp.float32),
        ),
    )(W1, b1.reshape(1, HIDDEN), W2, b2.reshape(1, LABELS))


def _tc_body(x_ref, wct_ref, bc_ref, o_ref):
    logits = lax.dot_general(
        x_ref[...], wct_ref[...], (((1,), (1,)), ((), ())),
        preferred_element_type=jnp.float32) + bc_ref[...]
    m = jnp.max(logits, axis=1, keepdims=True)
    s = logits - m
    o_ref[...] = s - jnp.log(jnp.sum(jnp.exp(s), axis=1, keepdims=True))


def _tc_body_alias(prev_ref, x_ref, wct_ref, bc_ref, o_ref):
    del prev_ref
    _tc_body(x_ref, wct_ref, bc_ref, o_ref)


def _tc_mlp_slice(prev, seq_slice, wct, bc, slice_idx):
    nblk = SB // BB
    x_spec = pl.BlockSpec((BB, EMBED), lambda j: (j, 0))
    w_spec = pl.BlockSpec((LABELS, EMBED), lambda j: (0, 0))
    b_spec = pl.BlockSpec((1, LABELS), lambda j: (0, 0))
    out_spec = pl.BlockSpec(
        (BB, LABELS), lambda j, s=slice_idx: (s * nblk + j, 0))
    out_shape = jax.ShapeDtypeStruct((B, LABELS), jnp.float32)
    if prev is None:
        return pl.pallas_call(
            _tc_body,
            grid=(nblk,),
            in_specs=[x_spec, w_spec, b_spec],
            out_specs=out_spec,
            out_shape=out_shape,
        )(seq_slice, wct, bc)
    return pl.pallas_call(
        _tc_body_alias,
        grid=(nblk,),
        in_specs=[
            pl.BlockSpec(memory_space=pltpu.MemorySpace.HBM),
            x_spec, w_spec, b_spec,
        ],
        out_specs=out_spec,
        out_shape=out_shape,
        input_output_aliases={0: 0},
    )(prev, seq_slice, wct, bc)


@jax.jit
def kernel(input_ids, seq_len, emb, W1, b1, W2, b2):
    ids2 = input_ids.astype(jnp.int32).reshape(B // CB, ROWS)
    wct, bc = _collapse(W1, b1, W2, b2)
    rows_per_slice = SB // CB

    def sc(i):
        return _sc_gather_pool(
            emb, lax.slice(ids2, (i * rows_per_slice, 0),
                           ((i + 1) * rows_per_slice, ROWS)))

    seq_prev = sc(0)
    out = None
    for i in range(1, NSLICE):
        seq_next = sc(i)
        out = _tc_mlp_slice(out, seq_prev, wct, bc, i - 1)
        seq_prev = seq_next
    return _tc_mlp_slice(out, seq_prev, wct, bc, NSLICE - 1)
